# baseline (speedup 1.0000x reference)
import jax
import jax.numpy as jnp
from jax import lax
from jax.experimental import pallas as pl
from jax.experimental.pallas import tpu as pltpu
from jax.experimental.pallas import tpu_sc as plsc

N = 10000
K = 16
D = 512
DH = D // 4

NC = 2
NS = 16
L = 16
NW = NC * NS

RPW = 320
NP = NW * RPW
SPS = NP // NS

W_T1 = 2 * DH
C1 = (L + DH) // L
C2 = DH // L

BR1 = 4
BR2 = 4


def _mesh():
  return plsc.VectorSubcoreMesh(
      core_axis_name="c", subcore_axis_name="s", num_cores=NC, num_subcores=NS)


def _gather_mean_body(br, n_chunks, split_first):
  n_batches = RPW // br
  n_groups = n_batches // 2
  n_groups3 = n_batches // 3
  n_streams = (br * K + 127) // 128

  def body(table_hbm, knn_hbm, *rest):
    if split_first:
      (s_out, a_out, idx_all, rows0_v, rows1_v, rows2_v,
       outs_v, outa_v, sem0, sem1, sem2) = rest
      table = table_hbm
    else:
      (b_out, a_sp, idx_all, rows0_v, rows1_v, outb_v, sem0, sem1) = rest
      table = a_sp
    sid = lax.axis_index("s")
    wid = sid * NC + lax.axis_index("c")
    base = wid * RPW

    pltpu.sync_copy(knn_hbm.at[pl.ds(base * K, RPW * K)], idx_all)

    if not split_first:
      bounce = rows0_v.at[pl.ds(0, br * K)]
      def st(i, carry):
        off = sid * SPS + i * (br * K)
        pltpu.sync_copy(table_hbm.at[pl.ds(off, br * K)], bounce)
        pltpu.sync_copy(bounce, a_sp.at[pl.ds(off, br * K)])
        return carry
      lax.fori_loop(0, SPS // (br * K), st, 0)
      plsc.subcore_barrier()

    half = br * K // 2

    def stage_and_fire(b, rows_v, sem):
      for j in range(2):
        pltpu.async_copy(
            table.at[idx_all.at[pl.ds(b * br * K + j * half, half)]],
            rows_v.at[pl.ds(j * half, half)], sem)

    def drain(b, rows_v, sem):
      for j in range(2):
        pltpu.make_async_copy(
            table.at[idx_all.at[pl.ds(b * br * K + j * half, half)]],
            rows_v.at[pl.ds(j * half, half)], sem).wait()

    def process(b, rows_v):
      row0 = base + b * br
      for r in range(br):
        for c in range(n_chunks):
          acc = rows_v[r * K, pl.ds(c * L, L)]
          for k in range(1, K):
            acc = acc + rows_v[r * K + k, pl.ds(c * L, L)]
          acc = acc * (1.0 / K)
          if split_first:
            if c == 0:
              outs_v[r, :] = acc
            else:
              outa_v[r, pl.ds((c - 1) * L, L)] = acc
          else:
            outb_v[r, pl.ds(c * L, L)] = acc
      if split_first:
        pltpu.sync_copy(outa_v, a_out.at[pl.ds(row0, br)])
        pltpu.sync_copy(outs_v, s_out.at[pl.ds(row0, br)])
      else:
        pltpu.sync_copy(outb_v, b_out.at[pl.ds(row0, br)])

    if split_first:
      stage_and_fire(0, rows0_v, sem0)
      stage_and_fire(1, rows1_v, sem1)

      def group3(g, carry):
        b0 = g * 3
        stage_and_fire(b0 + 2, rows2_v, sem2)
        drain(b0, rows0_v, sem0)
        process(b0, rows0_v)
        @pl.when(g < n_groups3 - 1)
        def _():
          stage_and_fire(b0 + 3, rows0_v, sem0)
        drain(b0 + 1, rows1_v, sem1)
        process(b0 + 1, rows1_v)
        @pl.when(g < n_groups3 - 1)
        def _():
          stage_and_fire(b0 + 4, rows1_v, sem1)
        drain(b0 + 2, rows2_v, sem2)
        process(b0 + 2, rows2_v)
        return carry

      lax.fori_loop(0, n_groups3, group3, 0)
      for b in range(n_groups3 * 3, n_batches):
        stage_and_fire(b, rows0_v, sem0)
        drain(b, rows0_v, sem0)
        process(b, rows0_v)
    else:
      stage_and_fire(0, rows0_v, sem0)

      def group(g, carry):
        b0 = g * 2
        stage_and_fire(b0 + 1, rows1_v, sem1)
        drain(b0, rows0_v, sem0)
        process(b0, rows0_v)
        @pl.when(g < n_groups - 1)
        def _():
          stage_and_fire(b0 + 2, rows0_v, sem0)
        drain(b0 + 1, rows1_v, sem1)
        process(b0 + 1, rows1_v)
        return carry

      lax.fori_loop(0, n_groups, group, 0)

  return body


def _make_pass1():
  f32 = jnp.float32
  scratch = [
      pltpu.VMEM((RPW * K,), jnp.int32),
      pltpu.VMEM((BR1 * K, W_T1), f32),
      pltpu.VMEM((BR1 * K, W_T1), f32),
      pltpu.VMEM((BR1 * K, W_T1), f32),
      pltpu.VMEM((BR1, L), f32),
      pltpu.VMEM((BR1, DH), f32),
      pltpu.SemaphoreType.DMA,
      pltpu.SemaphoreType.DMA,
      pltpu.SemaphoreType.DMA,
  ]
  return pl.kernel(
      _gather_mean_body(BR1, C1, True),
      out_type=(jax.ShapeDtypeStruct((NP, L), f32),
                jax.ShapeDtypeStruct((NP, DH), f32)),
      mesh=_mesh(),
      scratch_types=scratch,
      name="sc_gather_mean_pass1",
  )


def _make_pass2():
  f32 = jnp.float32
  scratch = [
      pltpu.VMEM_SHARED((NP, DH), f32),
      pltpu.VMEM((RPW * K,), jnp.int32),
      pltpu.VMEM((BR2 * K, DH), f32),
      pltpu.VMEM((BR2 * K, DH), f32),
      pltpu.VMEM((BR2, DH), f32),
      pltpu.SemaphoreType.DMA,
      pltpu.SemaphoreType.DMA,
  ]
  return pl.kernel(
      _gather_mean_body(BR2, C2, False),
      out_type=jax.ShapeDtypeStruct((NP, DH), f32),
      mesh=_mesh(),
      scratch_types=scratch,
      name="sc_gather_mean_pass2",
  )


_ROWS_BLK = 400
_GRID = N // _ROWS_BLK


def _tca_body(x_ref, gf_ref, wx_ref, wg_ref, wg2_ref, bc1_ref, bl2_ref,
              t1_ref, g2_ref):
  gf = gf_ref[...]
  t1_ref[...] = (
      jnp.dot(gf, wg_ref[...], preferred_element_type=jnp.float32)
      + jnp.dot(x_ref[...], wx_ref[...], preferred_element_type=jnp.float32)
      + bc1_ref[...])
  g2_ref[...] = (
      jnp.dot(gf, wg2_ref[...], preferred_element_type=jnp.float32)
      + bl2_ref[...])


def _tcb_body(g2_ref, s_ref, b2_ref, x_ref, w2a_ref, ws_ref, w2c_ref,
              bias_ref, out_ref):
  out_ref[...] = (
      jnp.dot(g2_ref[...], w2a_ref[...], preferred_element_type=jnp.float32)
      + jnp.dot(s_ref[...], ws_ref[...], preferred_element_type=jnp.float32)
      + jnp.dot(b2_ref[...], w2c_ref[...], preferred_element_type=jnp.float32)
      + bias_ref[...] + x_ref[...])


def _row_blocked(width):
  return pl.BlockSpec((_ROWS_BLK, width), lambda i: (i, 0))


def _whole(shape):
  return pl.BlockSpec(shape, lambda i: (0,) * len(shape))


def kernel(inputs, geometric_features, knn, W1, b1, Wl1, bl1, Wl2, bl2,
           W2, b2):
  f32 = jnp.float32
  gf_flat = geometric_features.reshape(N, K * 4)

  sel = jnp.tile(jnp.eye(4, dtype=f32), (K, 1)) * (1.0 / K)
  wg = jnp.concatenate([sel, jnp.zeros((K * 4, W_T1 - 4), f32)], axis=1)
  wx = jnp.concatenate(
      [jnp.zeros((D, L), f32), W1, jnp.zeros((D, W_T1 - L - DH), f32)],
      axis=1)
  bc1 = jnp.concatenate(
      [jnp.zeros((L,), f32), b1, jnp.zeros((W_T1 - L - DH,), f32)])[None, :]
  wg2 = jnp.tile(Wl2, (K, 1)) * (1.0 / K)

  w2a = W2[0:2 * DH]
  w2b = W2[2 * DH:3 * DH]
  w2c = W2[3 * DH:4 * DH]
  ws = jnp.concatenate([Wl1 @ w2b, jnp.zeros((L - 4, D), f32)], axis=0)
  bias_tot = (b2 + bl1 @ w2b)[None, :]

  knn32 = knn.astype(jnp.int32)
  knn_flat = jnp.pad(knn32, ((0, NP - N), (0, 0))).reshape(-1)

  t1, g2 = pl.pallas_call(
      _tca_body,
      grid=(_GRID,),
      in_specs=[
          _row_blocked(D), _row_blocked(K * 4),
          _whole((D, W_T1)), _whole((K * 4, W_T1)), _whole((K * 4, 2 * DH)),
          _whole((1, W_T1)), _whole((1, 2 * DH)),
      ],
      out_specs=[_row_blocked(W_T1), _row_blocked(2 * DH)],
      out_shape=(jax.ShapeDtypeStruct((N, W_T1), f32),
                 jax.ShapeDtypeStruct((N, 2 * DH), f32)),
  )(inputs, gf_flat, wx, wg, wg2, bc1, bl2[None, :])

  s_pad, a_pad = _make_pass1()(t1, knn_flat)
  b2g_pad = _make_pass2()(a_pad, knn_flat)

  out = pl.pallas_call(
      _tcb_body,
      grid=(_GRID,),
      in_specs=[
          _row_blocked(2 * DH), _row_blocked(L), _row_blocked(DH),
          _row_blocked(D),
          _whole((2 * DH, D)), _whole((L, D)), _whole((DH, D)),
          _whole((1, D)),
      ],
      out_specs=_row_blocked(D),
      out_shape=jax.ShapeDtypeStruct((N, D), f32),
  )(g2, s_pad, b2g_pad, inputs, w2a, ws, w2c, bias_tot)
  return out


# speedup vs baseline: 1.0126x; 1.0126x over previous
import jax
import jax.numpy as jnp
from jax import lax
from jax.experimental import pallas as pl
from jax.experimental.pallas import tpu as pltpu
from jax.experimental.pallas import tpu_sc as plsc

N = 10000
K = 16
D = 512
DH = D // 4

NC = 2
NS = 16
L = 16
NW = NC * NS

RPW = 320
NP = NW * RPW
SPS = NP // NS

W_T1 = 2 * DH
C1 = (L + DH) // L
C2 = DH // L

BR1 = 4
BR2 = 4


def _mesh():
  return plsc.VectorSubcoreMesh(
      core_axis_name="c", subcore_axis_name="s", num_cores=NC, num_subcores=NS)


def _gather_mean_body(br, n_chunks, split_first):
  n_batches = RPW // br
  n_groups = n_batches // 2
  n_groups3 = n_batches // 3
  n_streams = (br * K + 127) // 128

  def body(table_hbm, knn_hbm, *rest):
    if split_first:
      (s_out, a_out, idx_all, rows0_v, rows1_v, rows2_v,
       outs_v, outa_v, sem0, sem1, sem2) = rest
      table = table_hbm
    else:
      (b_out, a_sp, idx_all, rows0_v, rows1_v, outb_v, sem0, sem1) = rest
      table = a_sp
    sid = lax.axis_index("s")
    wid = sid * NC + lax.axis_index("c")
    base = wid * RPW

    pltpu.sync_copy(knn_hbm.at[pl.ds(base * K, RPW * K)], idx_all)

    if not split_first:
      bounce = rows0_v.at[pl.ds(0, br * K)]
      def st(i, carry):
        off = sid * SPS + i * (br * K)
        pltpu.sync_copy(table_hbm.at[pl.ds(off, br * K)], bounce)
        pltpu.sync_copy(bounce, a_sp.at[pl.ds(off, br * K)])
        return carry
      lax.fori_loop(0, SPS // (br * K), st, 0)
      plsc.subcore_barrier()

    def stage_and_fire(b, rows_v, sem):
      for j in range(n_streams):
        pltpu.async_copy(
            table.at[idx_all.at[pl.ds(b * br * K + j * 128, br * K)]],
            rows_v.at[pl.ds(j * 128, br * K)], sem)

    def drain(b, rows_v, sem):
      for j in range(n_streams):
        pltpu.make_async_copy(
            table.at[idx_all.at[pl.ds(b * br * K + j * 128, br * K)]],
            rows_v.at[pl.ds(j * 128, br * K)], sem).wait()

    def process(b, rows_v):
      row0 = base + b * br
      for r in range(br):
        for c in range(n_chunks):
          acc = rows_v[r * K, pl.ds(c * L, L)]
          for k in range(1, K):
            acc = acc + rows_v[r * K + k, pl.ds(c * L, L)]
          acc = acc * (1.0 / K)
          if split_first:
            if c == 0:
              outs_v[r, :] = acc
            else:
              outa_v[r, pl.ds((c - 1) * L, L)] = acc
          else:
            outb_v[r, pl.ds(c * L, L)] = acc
      if split_first:
        pltpu.sync_copy(outa_v, a_out.at[pl.ds(row0, br)])
        pltpu.sync_copy(outs_v, s_out.at[pl.ds(row0, br)])
      else:
        pltpu.sync_copy(outb_v, b_out.at[pl.ds(row0, br)])

    if split_first:
      stage_and_fire(0, rows0_v, sem0)
      stage_and_fire(1, rows1_v, sem1)

      def group3(g, carry):
        b0 = g * 3
        stage_and_fire(b0 + 2, rows2_v, sem2)
        drain(b0, rows0_v, sem0)
        process(b0, rows0_v)
        @pl.when(g < n_groups3 - 1)
        def _():
          stage_and_fire(b0 + 3, rows0_v, sem0)
        drain(b0 + 1, rows1_v, sem1)
        process(b0 + 1, rows1_v)
        @pl.when(g < n_groups3 - 1)
        def _():
          stage_and_fire(b0 + 4, rows1_v, sem1)
        drain(b0 + 2, rows2_v, sem2)
        process(b0 + 2, rows2_v)
        return carry

      lax.fori_loop(0, n_groups3, group3, 0)
      for b in range(n_groups3 * 3, n_batches):
        stage_and_fire(b, rows0_v, sem0)
        drain(b, rows0_v, sem0)
        process(b, rows0_v)
    else:
      stage_and_fire(0, rows0_v, sem0)

      def group(g, carry):
        b0 = g * 2
        stage_and_fire(b0 + 1, rows1_v, sem1)
        drain(b0, rows0_v, sem0)
        process(b0, rows0_v)
        @pl.when(g < n_groups - 1)
        def _():
          stage_and_fire(b0 + 2, rows0_v, sem0)
        drain(b0 + 1, rows1_v, sem1)
        process(b0 + 1, rows1_v)
        return carry

      lax.fori_loop(0, n_groups, group, 0)

  return body


def _make_pass1():
  f32 = jnp.float32
  scratch = [
      pltpu.VMEM((RPW * K,), jnp.int32),
      pltpu.VMEM((BR1 * K, W_T1), f32),
      pltpu.VMEM((BR1 * K, W_T1), f32),
      pltpu.VMEM((BR1 * K, W_T1), f32),
      pltpu.VMEM((BR1, L), f32),
      pltpu.VMEM((BR1, DH), f32),
      pltpu.SemaphoreType.DMA,
      pltpu.SemaphoreType.DMA,
      pltpu.SemaphoreType.DMA,
  ]
  return pl.kernel(
      _gather_mean_body(BR1, C1, True),
      out_type=(jax.ShapeDtypeStruct((NP, L), f32),
                jax.ShapeDtypeStruct((NP, DH), f32)),
      mesh=_mesh(),
      scratch_types=scratch,
      name="sc_gather_mean_pass1",
  )


def _make_pass2():
  f32 = jnp.float32
  scratch = [
      pltpu.VMEM_SHARED((NP, DH), f32),
      pltpu.VMEM((RPW * K,), jnp.int32),
      pltpu.VMEM((BR2 * K, DH), f32),
      pltpu.VMEM((BR2 * K, DH), f32),
      pltpu.VMEM((BR2, DH), f32),
      pltpu.SemaphoreType.DMA,
      pltpu.SemaphoreType.DMA,
  ]
  return pl.kernel(
      _gather_mean_body(BR2, C2, False),
      out_type=jax.ShapeDtypeStruct((NP, DH), f32),
      mesh=_mesh(),
      scratch_types=scratch,
      name="sc_gather_mean_pass2",
  )


_ROWS_BLK = 400
_GRID = N // _ROWS_BLK


def _tca_body(x_ref, gf_ref, wx_ref, wg_ref, wg2_ref, bc1_ref, bl2_ref,
              t1_ref, g2_ref):
  gf = gf_ref[...]
  t1_ref[...] = (
      jnp.dot(gf, wg_ref[...], preferred_element_type=jnp.float32)
      + jnp.dot(x_ref[...], wx_ref[...], preferred_element_type=jnp.float32)
      + bc1_ref[...])
  g2_ref[...] = (
      jnp.dot(gf, wg2_ref[...], preferred_element_type=jnp.float32)
      + bl2_ref[...])


def _tcb_body(g2_ref, s_ref, b2_ref, x_ref, w2a_ref, ws_ref, w2c_ref,
              bias_ref, out_ref):
  out_ref[...] = (
      jnp.dot(g2_ref[...], w2a_ref[...], preferred_element_type=jnp.float32)
      + jnp.dot(s_ref[...], ws_ref[...], preferred_element_type=jnp.float32)
      + jnp.dot(b2_ref[...], w2c_ref[...], preferred_element_type=jnp.float32)
      + bias_ref[...] + x_ref[...])


def _row_blocked(width):
  return pl.BlockSpec((_ROWS_BLK, width), lambda i: (i, 0))


def _whole(shape):
  return pl.BlockSpec(shape, lambda i: (0,) * len(shape))


def kernel(inputs, geometric_features, knn, W1, b1, Wl1, bl1, Wl2, bl2,
           W2, b2):
  f32 = jnp.float32
  gf_flat = geometric_features.reshape(N, K * 4)

  sel = jnp.tile(jnp.eye(4, dtype=f32), (K, 1)) * (1.0 / K)
  wg = jnp.concatenate([sel, jnp.zeros((K * 4, W_T1 - 4), f32)], axis=1)
  wx = jnp.concatenate(
      [jnp.zeros((D, L), f32), W1, jnp.zeros((D, W_T1 - L - DH), f32)],
      axis=1)
  bc1 = jnp.concatenate(
      [jnp.zeros((L,), f32), b1, jnp.zeros((W_T1 - L - DH,), f32)])[None, :]
  wg2 = jnp.tile(Wl2, (K, 1)) * (1.0 / K)

  w2a = W2[0:2 * DH]
  w2b = W2[2 * DH:3 * DH]
  w2c = W2[3 * DH:4 * DH]
  ws = jnp.concatenate([Wl1 @ w2b, jnp.zeros((L - 4, D), f32)], axis=0)
  bias_tot = (b2 + bl1 @ w2b)[None, :]

  knn32 = knn.astype(jnp.int32)
  knn_flat = jnp.pad(knn32, ((0, NP - N), (0, 0))).reshape(-1)

  t1, g2 = pl.pallas_call(
      _tca_body,
      grid=(_GRID,),
      in_specs=[
          _row_blocked(D), _row_blocked(K * 4),
          _whole((D, W_T1)), _whole((K * 4, W_T1)), _whole((K * 4, 2 * DH)),
          _whole((1, W_T1)), _whole((1, 2 * DH)),
      ],
      out_specs=[_row_blocked(W_T1), _row_blocked(2 * DH)],
      out_shape=(jax.ShapeDtypeStruct((N, W_T1), f32),
                 jax.ShapeDtypeStruct((N, 2 * DH), f32)),
  )(inputs, gf_flat, wx, wg, wg2, bc1, bl2[None, :])

  s_pad, a_pad = _make_pass1()(t1, knn_flat)
  b2g_pad = _make_pass2()(a_pad, knn_flat)

  out = pl.pallas_call(
      _tcb_body,
      grid=(_GRID,),
      in_specs=[
          _row_blocked(2 * DH), _row_blocked(L), _row_blocked(DH),
          _row_blocked(D),
          _whole((2 * DH, D)), _whole((L, D)), _whole((DH, D)),
          _whole((1, D)),
      ],
      out_specs=_row_blocked(D),
      out_shape=jax.ShapeDtypeStruct((N, D), f32),
  )(g2, s_pad, b2g_pad, inputs, w2a, ws, w2c, bias_tot)
  return out


# speedup vs baseline: 1.2133x; 1.1982x over previous
import jax
import jax.numpy as jnp
from jax import lax
from jax.experimental import pallas as pl
from jax.experimental.pallas import tpu as pltpu
from jax.experimental.pallas import tpu_sc as plsc

N = 10000
K = 16
D = 512
DH = D // 4

NC = 2
NS = 16
L = 16
NW = NC * NS

RPW = 320
NP = NW * RPW
SPS = NP // NS

W_T1 = 2 * DH
C1 = (L + DH) // L
C2 = DH // L

BR1 = 4
BR2 = 4


def _mesh():
  return plsc.VectorSubcoreMesh(
      core_axis_name="c", subcore_axis_name="s", num_cores=NC, num_subcores=NS)


def _gather_mean_body(br, n_chunks, split_first):
  n_batches = RPW // br
  n_groups = n_batches // 2
  n_groups3 = n_batches // 3
  n_streams = (br * K + 127) // 128

  def body(table_hbm, knn_hbm, *rest):
    if split_first:
      (s_out, a_out, idx_all, rows0_v, rows1_v, rows2_v,
       outs_v, outa_v, sem0, sem1, sem2) = rest
      table = table_hbm
    else:
      (b_out, a_sp, idx_all, rows0_v, rows1_v, outb_v, sem0, sem1) = rest
      table = a_sp
    sid = lax.axis_index("s")
    wid = sid * NC + lax.axis_index("c")
    base = wid * RPW

    pltpu.sync_copy(knn_hbm.at[pl.ds(base * K, RPW * K)], idx_all)

    if not split_first:
      bounce = rows0_v.at[pl.ds(0, br * K)]
      def st(i, carry):
        off = sid * SPS + i * (br * K)
        pltpu.sync_copy(table_hbm.at[pl.ds(off, br * K)], bounce)
        pltpu.sync_copy(bounce, a_sp.at[pl.ds(off, br * K)])
        return carry
      lax.fori_loop(0, SPS // (br * K), st, 0)
      plsc.subcore_barrier()

    def stage_and_fire(b, rows_v, sem):
      for j in range(n_streams):
        pltpu.async_copy(
            table.at[idx_all.at[pl.ds(b * br * K + j * 128, br * K)]],
            rows_v.at[pl.ds(j * 128, br * K)], sem)

    def drain(b, rows_v, sem):
      for j in range(n_streams):
        pltpu.make_async_copy(
            table.at[idx_all.at[pl.ds(b * br * K + j * 128, br * K)]],
            rows_v.at[pl.ds(j * 128, br * K)], sem).wait()

    def process(b, rows_v):
      row0 = base + b * br
      for r in range(br):
        for c in range(n_chunks):
          acc = rows_v[r * K, pl.ds(c * L, L)]
          for k in range(1, K):
            acc = acc + rows_v[r * K + k, pl.ds(c * L, L)]
          acc = acc * (1.0 / K)
          if split_first:
            if c == 0:
              outs_v[r, :] = acc
            else:
              outa_v[r, pl.ds((c - 1) * L, L)] = acc
          else:
            outb_v[r, pl.ds(c * L, L)] = acc
      if split_first:
        pltpu.sync_copy(outa_v, a_out.at[pl.ds(row0, br)])
        pltpu.sync_copy(outs_v, s_out.at[pl.ds(row0, br)])
      else:
        pltpu.sync_copy(outb_v, b_out.at[pl.ds(row0, br)])

    if split_first:
      stage_and_fire(0, rows0_v, sem0)
      stage_and_fire(1, rows1_v, sem1)

      def group3(g, carry):
        b0 = g * 3
        stage_and_fire(b0 + 2, rows2_v, sem2)
        drain(b0, rows0_v, sem0)
        process(b0, rows0_v)
        @pl.when(g < n_groups3 - 1)
        def _():
          stage_and_fire(b0 + 3, rows0_v, sem0)
        drain(b0 + 1, rows1_v, sem1)
        process(b0 + 1, rows1_v)
        @pl.when(g < n_groups3 - 1)
        def _():
          stage_and_fire(b0 + 4, rows1_v, sem1)
        drain(b0 + 2, rows2_v, sem2)
        process(b0 + 2, rows2_v)
        return carry

      lax.fori_loop(0, n_groups3, group3, 0)
      for b in range(n_groups3 * 3, n_batches):
        stage_and_fire(b, rows0_v, sem0)
        drain(b, rows0_v, sem0)
        process(b, rows0_v)
    else:
      stage_and_fire(0, rows0_v, sem0)

      def group(g, carry):
        b0 = g * 2
        stage_and_fire(b0 + 1, rows1_v, sem1)
        drain(b0, rows0_v, sem0)
        process(b0, rows0_v)
        @pl.when(g < n_groups - 1)
        def _():
          stage_and_fire(b0 + 2, rows0_v, sem0)
        drain(b0 + 1, rows1_v, sem1)
        process(b0 + 1, rows1_v)
        return carry

      lax.fori_loop(0, n_groups, group, 0)

  return body


def _make_pass1():
  f32 = jnp.float32
  scratch = [
      pltpu.VMEM((RPW * K,), jnp.int32),
      pltpu.VMEM((BR1 * K, W_T1), f32),
      pltpu.VMEM((BR1 * K, W_T1), f32),
      pltpu.VMEM((BR1 * K, W_T1), f32),
      pltpu.VMEM((BR1, L), f32),
      pltpu.VMEM((BR1, DH), f32),
      pltpu.SemaphoreType.DMA,
      pltpu.SemaphoreType.DMA,
      pltpu.SemaphoreType.DMA,
  ]
  return pl.kernel(
      _gather_mean_body(BR1, C1, True),
      out_type=(jax.ShapeDtypeStruct((NP, L), f32),
                jax.ShapeDtypeStruct((NP, DH), f32)),
      mesh=_mesh(),
      scratch_types=scratch,
      name="sc_gather_mean_pass1",
  )


def _make_pass2():
  f32 = jnp.float32
  scratch = [
      pltpu.VMEM_SHARED((NP, DH), f32),
      pltpu.VMEM((RPW * K,), jnp.int32),
      pltpu.VMEM((BR2 * K, DH), f32),
      pltpu.VMEM((BR2 * K, DH), f32),
      pltpu.VMEM((BR2, DH), f32),
      pltpu.SemaphoreType.DMA,
      pltpu.SemaphoreType.DMA,
  ]
  return pl.kernel(
      _gather_mean_body(BR2, C2, False),
      out_type=jax.ShapeDtypeStruct((NP, DH), f32),
      mesh=_mesh(),
      scratch_types=scratch,
      name="sc_gather_mean_pass2",
  )


_ROWS_BLK = 400
_GRID = N // _ROWS_BLK


def _tca_body(x_ref, gf_ref, w1_ref, wg_ref, wg2_ref, b1_ref, bl2_ref,
              h0_ref, gfm_ref, g2_ref):
  gf = gf_ref[...]
  h0_ref[...] = (
      jnp.dot(x_ref[...], w1_ref[...], preferred_element_type=jnp.float32)
      + b1_ref[...])
  gfm_ref[...] = jnp.dot(gf, wg_ref[...], preferred_element_type=jnp.float32)
  g2_ref[...] = (
      jnp.dot(gf, wg2_ref[...], preferred_element_type=jnp.float32)
      + bl2_ref[...])


def _tcb_body(g2_ref, s_ref, b2_ref, x_ref, w2a_ref, ws_ref, w2c_ref,
              bias_ref, out_ref):
  out_ref[...] = (
      jnp.dot(g2_ref[...], w2a_ref[...], preferred_element_type=jnp.float32)
      + jnp.dot(s_ref[...], ws_ref[...], preferred_element_type=jnp.float32)
      + jnp.dot(b2_ref[...], w2c_ref[...], preferred_element_type=jnp.float32)
      + bias_ref[...] + x_ref[...])


def _row_blocked(width):
  return pl.BlockSpec((_ROWS_BLK, width), lambda i: (i, 0))


def _whole(shape):
  return pl.BlockSpec(shape, lambda i: (0,) * len(shape))


def kernel(inputs, geometric_features, knn, W1, b1, Wl1, bl1, Wl2, bl2,
           W2, b2):
  f32 = jnp.float32
  gf_flat = geometric_features.reshape(N, K * 4)

  sel = jnp.tile(jnp.eye(4, dtype=f32), (K, 1)) * (1.0 / K)
  wg = jnp.concatenate([sel, jnp.zeros((K * 4, DH - 4), f32)], axis=1)
  wg2 = jnp.tile(Wl2, (K, 1)) * (1.0 / K)

  w2a = W2[0:2 * DH]
  w2b = W2[2 * DH:3 * DH]
  w2c = W2[3 * DH:4 * DH]
  ws = jnp.concatenate([Wl1 @ w2b, jnp.zeros((DH - 4, D), f32)], axis=0)
  bias_tot = (b2 + bl1 @ w2b)[None, :]

  knn32 = knn.astype(jnp.int32)
  knn_flat = jnp.pad(knn32, ((0, NP - N), (0, 0))).reshape(-1)

  h0, gfm, g2 = pl.pallas_call(
      _tca_body,
      grid=(_GRID,),
      in_specs=[
          _row_blocked(D), _row_blocked(K * 4),
          _whole((D, DH)), _whole((K * 4, DH)), _whole((K * 4, 2 * DH)),
          _whole((1, DH)), _whole((1, 2 * DH)),
      ],
      out_specs=[_row_blocked(DH), _row_blocked(DH), _row_blocked(2 * DH)],
      out_shape=(jax.ShapeDtypeStruct((N, DH), f32),
                 jax.ShapeDtypeStruct((N, DH), f32),
                 jax.ShapeDtypeStruct((N, 2 * DH), f32)),
  )(inputs, gf_flat, W1, wg, wg2, b1[None, :], bl2[None, :])

  h0_pad = jnp.pad(h0, ((0, NP - N), (0, 0)))
  gfm_pad = jnp.pad(gfm, ((0, NP - N), (0, 0)))

  gm = _make_pass2()
  s_pad = gm(gfm_pad, knn_flat)
  a_pad = gm(h0_pad, knn_flat)
  b2g_pad = gm(a_pad, knn_flat)

  out = pl.pallas_call(
      _tcb_body,
      grid=(_GRID,),
      in_specs=[
          _row_blocked(2 * DH), _row_blocked(DH), _row_blocked(DH),
          _row_blocked(D),
          _whole((2 * DH, D)), _whole((DH, D)), _whole((DH, D)),
          _whole((1, D)),
      ],
      out_specs=_row_blocked(D),
      out_shape=jax.ShapeDtypeStruct((N, D), f32),
  )(g2, s_pad, b2g_pad, inputs, w2a, ws, w2c, bias_tot)
  return out


# speedup vs baseline: 1.2235x; 1.0084x over previous
import jax
import jax.numpy as jnp
from jax import lax
from jax.experimental import pallas as pl
from jax.experimental.pallas import tpu as pltpu
from jax.experimental.pallas import tpu_sc as plsc

N = 10000
K = 16
D = 512
DH = D // 4

NC = 2
NS = 16
L = 16
NW = NC * NS

RPW = 320
NP = NW * RPW
SPS = NP // NS

W_T1 = 2 * DH
C1 = (L + DH) // L
C2 = DH // L

BR1 = 4
BR2 = 4


def _mesh():
  return plsc.VectorSubcoreMesh(
      core_axis_name="c", subcore_axis_name="s", num_cores=NC, num_subcores=NS)


def _gather_mean_body(br, n_chunks, split_first):
  n_batches = RPW // br
  n_groups = n_batches // 2
  n_groups3 = n_batches // 3
  n_streams = (br * K + 127) // 128

  def body(table_hbm, knn_hbm, *rest):
    if split_first:
      (s_out, a_out, idx_all, rows0_v, rows1_v, rows2_v,
       outs_v, outa_v, sem0, sem1, sem2) = rest
      table = table_hbm
    else:
      (b_out, a_sp, idx_all, rows0_v, rows1_v, outb_v, sem0, sem1) = rest
      table = a_sp
    sid = lax.axis_index("s")
    wid = sid * NC + lax.axis_index("c")
    base = wid * RPW

    pltpu.sync_copy(knn_hbm.at[pl.ds(base * K, RPW * K)], idx_all)

    if not split_first:
      bounce = rows0_v.at[pl.ds(0, br * K)]
      def st(i, carry):
        off = sid * SPS + i * (br * K)
        pltpu.sync_copy(table_hbm.at[pl.ds(off, br * K)], bounce)
        pltpu.sync_copy(bounce, a_sp.at[pl.ds(off, br * K)])
        return carry
      lax.fori_loop(0, SPS // (br * K), st, 0)
      plsc.subcore_barrier()

    def stage_and_fire(b, rows_v, sem):
      for j in range(n_streams):
        pltpu.async_copy(
            table.at[idx_all.at[pl.ds(b * br * K + j * 128, br * K)]],
            rows_v.at[pl.ds(j * 128, br * K)], sem)

    def drain(b, rows_v, sem):
      for j in range(n_streams):
        pltpu.make_async_copy(
            table.at[idx_all.at[pl.ds(b * br * K + j * 128, br * K)]],
            rows_v.at[pl.ds(j * 128, br * K)], sem).wait()

    def process(b, rows_v):
      row0 = base + b * br
      for r in range(br):
        for c in range(n_chunks):
          acc = rows_v[r * K, pl.ds(c * L, L)]
          for k in range(1, K):
            acc = acc + rows_v[r * K + k, pl.ds(c * L, L)]
          acc = acc * (1.0 / K)
          if split_first:
            if c == 0:
              outs_v[r, :] = acc
            else:
              outa_v[r, pl.ds((c - 1) * L, L)] = acc
          else:
            outb_v[r, pl.ds(c * L, L)] = acc
      if split_first:
        pltpu.sync_copy(outa_v, a_out.at[pl.ds(row0, br)])
        pltpu.sync_copy(outs_v, s_out.at[pl.ds(row0, br)])
      else:
        pltpu.sync_copy(outb_v, b_out.at[pl.ds(row0, br)])

    if split_first:
      stage_and_fire(0, rows0_v, sem0)
      stage_and_fire(1, rows1_v, sem1)

      def group3(g, carry):
        b0 = g * 3
        stage_and_fire(b0 + 2, rows2_v, sem2)
        drain(b0, rows0_v, sem0)
        process(b0, rows0_v)
        @pl.when(g < n_groups3 - 1)
        def _():
          stage_and_fire(b0 + 3, rows0_v, sem0)
        drain(b0 + 1, rows1_v, sem1)
        process(b0 + 1, rows1_v)
        @pl.when(g < n_groups3 - 1)
        def _():
          stage_and_fire(b0 + 4, rows1_v, sem1)
        drain(b0 + 2, rows2_v, sem2)
        process(b0 + 2, rows2_v)
        return carry

      lax.fori_loop(0, n_groups3, group3, 0)
      for b in range(n_groups3 * 3, n_batches):
        stage_and_fire(b, rows0_v, sem0)
        drain(b, rows0_v, sem0)
        process(b, rows0_v)
    else:
      stage_and_fire(0, rows0_v, sem0)

      def group(g, carry):
        b0 = g * 2
        stage_and_fire(b0 + 1, rows1_v, sem1)
        drain(b0, rows0_v, sem0)
        process(b0, rows0_v)
        @pl.when(g < n_groups - 1)
        def _():
          stage_and_fire(b0 + 2, rows0_v, sem0)
        drain(b0 + 1, rows1_v, sem1)
        process(b0 + 1, rows1_v)
        return carry

      lax.fori_loop(0, n_groups, group, 0)

  return body


def _make_pass2():
  f32 = jnp.float32
  scratch = [
      pltpu.VMEM_SHARED((NP, DH), f32),
      pltpu.VMEM((RPW * K,), jnp.int32),
      pltpu.VMEM((BR2 * K, DH), f32),
      pltpu.VMEM((BR2 * K, DH), f32),
      pltpu.VMEM((BR2, DH), f32),
      pltpu.SemaphoreType.DMA,
      pltpu.SemaphoreType.DMA,
  ]
  return pl.kernel(
      _gather_mean_body(BR2, C2, False),
      out_type=jax.ShapeDtypeStruct((NP, DH), f32),
      mesh=_mesh(),
      scratch_types=scratch,
      name="sc_gather_mean_pass2",
  )


_ROWS_BLK = 400
_GRID = N // _ROWS_BLK


def _tca_body(x_ref, gf_ref, w1_ref, wg_ref, wg2_ref, b1_ref, bl2_ref,
              h0_ref, gfm_ref, g2_ref):
  gf = gf_ref[...]
  h0_ref[...] = (
      jnp.dot(x_ref[...], w1_ref[...], preferred_element_type=jnp.float32)
      + b1_ref[...])
  gfm_ref[...] = jnp.dot(gf, wg_ref[...], preferred_element_type=jnp.float32)
  g2_ref[...] = (
      jnp.dot(gf, wg2_ref[...], preferred_element_type=jnp.float32)
      + bl2_ref[...])


def _tcb_body(g2_ref, s_ref, b2_ref, x_ref, w2a_ref, ws_ref, w2c_ref,
              bias_ref, out_ref):
  out_ref[...] = (
      jnp.dot(g2_ref[...], w2a_ref[...], preferred_element_type=jnp.float32)
      + jnp.dot(s_ref[...], ws_ref[...], preferred_element_type=jnp.float32)
      + jnp.dot(b2_ref[...], w2c_ref[...], preferred_element_type=jnp.float32)
      + bias_ref[...] + x_ref[...])


def _row_blocked(width):
  return pl.BlockSpec((_ROWS_BLK, width), lambda i: (i, 0))


def _whole(shape):
  return pl.BlockSpec(shape, lambda i: (0,) * len(shape))


def kernel(inputs, geometric_features, knn, W1, b1, Wl1, bl1, Wl2, bl2,
           W2, b2):
  f32 = jnp.float32
  gf_flat = geometric_features.reshape(N, K * 4)

  sel = jnp.tile(jnp.eye(4, dtype=f32), (K, 1)) * (1.0 / K)
  wg = jnp.concatenate([sel, jnp.zeros((K * 4, DH - 4), f32)], axis=1)
  wg2 = jnp.tile(Wl2, (K, 1)) * (1.0 / K)

  w2a = W2[0:2 * DH]
  w2b = W2[2 * DH:3 * DH]
  w2c = W2[3 * DH:4 * DH]
  ws = jnp.concatenate([Wl1 @ w2b, jnp.zeros((DH - 4, D), f32)], axis=0)
  bias_tot = (b2 + bl1 @ w2b)[None, :]

  knn32 = knn.astype(jnp.int32)
  knn_flat = jnp.pad(knn32, ((0, NP - N), (0, 0))).reshape(-1)

  h0, gfm, g2 = pl.pallas_call(
      _tca_body,
      grid=(_GRID,),
      in_specs=[
          _row_blocked(D), _row_blocked(K * 4),
          _whole((D, DH)), _whole((K * 4, DH)), _whole((K * 4, 2 * DH)),
          _whole((1, DH)), _whole((1, 2 * DH)),
      ],
      out_specs=[_row_blocked(DH), _row_blocked(DH), _row_blocked(2 * DH)],
      out_shape=(jax.ShapeDtypeStruct((N, DH), f32),
                 jax.ShapeDtypeStruct((N, DH), f32),
                 jax.ShapeDtypeStruct((N, 2 * DH), f32)),
  )(inputs, gf_flat, W1, wg, wg2, b1[None, :], bl2[None, :])

  h0_pad = jnp.pad(h0, ((0, NP - N), (0, 0)))
  gfm_pad = jnp.pad(gfm, ((0, NP - N), (0, 0)))

  gm = _make_pass2()
  s_pad = gm(gfm_pad, knn_flat)
  a_pad = gm(h0_pad, knn_flat)
  b2g_pad = gm(a_pad, knn_flat)

  out = pl.pallas_call(
      _tcb_body,
      grid=(_GRID,),
      in_specs=[
          _row_blocked(2 * DH), _row_blocked(DH), _row_blocked(DH),
          _row_blocked(D),
          _whole((2 * DH, D)), _whole((DH, D)), _whole((DH, D)),
          _whole((1, D)),
      ],
      out_specs=_row_blocked(D),
      out_shape=jax.ShapeDtypeStruct((N, D), f32),
  )(g2, s_pad, b2g_pad, inputs, w2a, ws, w2c, bias_tot)
  return out
